# baseline (speedup 1.0000x reference)
import functools

import jax
import jax.numpy as jnp
from jax import lax
from jax.experimental import pallas as pl
from jax.experimental.pallas import tpu as pltpu
from jax.experimental.pallas import tpu_sc as plsc

D_HALF = 128
NUM_BINS = 50
R_MIN = -4000.0
R_MAX = 4000.0

NC, NS, L = 2, 16, 16
NW = NC * NS

N_COORD = 16 * 8192 * 2
C_PER_W = N_COORD // NW
G_ROWS = 128
NGRP = C_PER_W // G_ROWS
NBUF = 4
T_ROWS = 2 * NUM_BINS


def _sc_body(pos_hbm, table_hbm, out_hbm, coords_v, idx_v, table_v,
             rb0, rb1, rb2, rb3, gs0, gs1, gs2, gs3, ws0, ws1, ws2, ws3):
    rbufs = (rb0, rb1, rb2, rb3)
    gsems = (gs0, gs1, gs2, gs3)
    wsems = (ws0, ws1, ws2, ws3)

    wid = lax.axis_index("s") * NC + lax.axis_index("c")
    base = wid * C_PER_W

    sid = lax.axis_index("s")
    pltpu.sync_copy(pos_hbm.at[pl.ds(base, C_PER_W)], coords_v)
    pltpu.sync_copy(table_hbm, table_v.at[pl.ds(sid * T_ROWS, T_ROWS)])

    offs = (lax.iota(jnp.int32, L) % 2) * NUM_BINS + sid * T_ROWS

    @pl.loop(0, C_PER_W // L)
    def _(i):
        c = coords_v[pl.ds(i * L, L)]
        n = jnp.clip((c - R_MIN) / (R_MAX - R_MIN), 0.0, 1.0)
        idx_v[pl.ds(i * L, L)] = (n * float(NUM_BINS - 1)).astype(jnp.int32) + offs

    def idx_slice(g):
        return idx_v.at[pl.ds(g * G_ROWS, G_ROWS)]

    def out_slice(g):
        return out_hbm.at[pl.ds(base + g * G_ROWS, G_ROWS)]

    pltpu.async_copy(table_v.at[idx_slice(0)], rbufs[0], gsems[0])
    pltpu.async_copy(table_v.at[idx_slice(1)], rbufs[1], gsems[1])

    @pl.loop(0, NGRP, step=NBUF)
    def _(g0):
        for b in range(NBUF):
            g = g0 + b
            nb = (b + 2) % NBUF

            @pl.when(g + 2 < NGRP)
            def _():
                @pl.when(g >= 2)
                def _():
                    pltpu.make_async_copy(rbufs[nb], out_slice(0), wsems[nb]).wait()
                pltpu.async_copy(table_v.at[idx_slice(g + 2)], rbufs[nb], gsems[nb])

            pltpu.make_async_copy(
                table_v.at[idx_slice(g)], rbufs[b], gsems[b]).wait()
            pltpu.async_copy(rbufs[b], out_slice(g), wsems[b])

    for b in range(NBUF):
        pltpu.make_async_copy(rbufs[b], out_slice(0), wsems[b]).wait()


@jax.jit
def _pos_encode(pos_flat, table):
    mesh = plsc.VectorSubcoreMesh(
        core_axis_name="c", subcore_axis_name="s", num_cores=NC, num_subcores=NS)
    f = pl.kernel(
        _sc_body,
        out_type=jax.ShapeDtypeStruct((N_COORD, D_HALF), jnp.float32),
        name="pos_encode_sc",
        mesh=mesh,
        scratch_types=[
            pltpu.VMEM((C_PER_W,), jnp.float32),
            pltpu.VMEM((C_PER_W,), jnp.int32),
            pltpu.VMEM_SHARED((NS * 2 * NUM_BINS, D_HALF), jnp.float32),
            pltpu.VMEM((G_ROWS, D_HALF), jnp.float32),
            pltpu.VMEM((G_ROWS, D_HALF), jnp.float32),
            pltpu.VMEM((G_ROWS, D_HALF), jnp.float32),
            pltpu.VMEM((G_ROWS, D_HALF), jnp.float32),
            pltpu.SemaphoreType.DMA,
            pltpu.SemaphoreType.DMA,
            pltpu.SemaphoreType.DMA,
            pltpu.SemaphoreType.DMA,
            pltpu.SemaphoreType.DMA,
            pltpu.SemaphoreType.DMA,
            pltpu.SemaphoreType.DMA,
            pltpu.SemaphoreType.DMA,
        ],
    )
    return f(pos_flat, table)


def kernel(positions, x_embed, y_embed):
    table = jnp.concatenate([x_embed, y_embed], axis=0)
    pos_flat = positions.reshape(-1)
    out = _pos_encode(pos_flat, table)
    return out.reshape(positions.shape[0], positions.shape[1], 2 * D_HALF)


# speedup vs baseline: 1.1603x; 1.1603x over previous
import jax
import jax.numpy as jnp
from jax import lax
from jax.experimental import pallas as pl
from jax.experimental.pallas import tpu as pltpu
from jax.experimental.pallas import tpu_sc as plsc

D_HALF = 128
D = 2 * D_HALF
NUM_BINS = 50
R_MIN = -4000.0
R_MAX = 4000.0

NC, NS, L = 2, 16, 16
NW = NC * NS

N_POS = 16 * 8192
P_PER_W = N_POS // NW
CH = 16
NCH = P_PER_W // CH
NBUF = 8

PAIR_PAD = 2560
PB_ROWS = PAIR_PAD // NS
B_PAD = 512
Y_OFF = 256


def _sc_body(xc_hbm, yc_hbm, ct_hbm, out_hbm,
             xc_v, yc_v, kidx_v, bidx_v, hb, paired,
             rb0, rb1, rb2, rb3, rb4, rb5, rb6, rb7,
             gs0, gs1, gs2, gs3, gs4, gs5, gs6, gs7,
             ws0, ws1, ws2, ws3, ws4, ws5, ws6, ws7):
    rbufs = (rb0, rb1, rb2, rb3, rb4, rb5, rb6, rb7)
    gsems = (gs0, gs1, gs2, gs3, gs4, gs5, gs6, gs7)
    wsems = (ws0, ws1, ws2, ws3, ws4, ws5, ws6, ws7)

    cid = lax.axis_index("c")
    sid = lax.axis_index("s")
    wid = sid * NC + cid
    base = wid * P_PER_W

    k0 = sid * PB_ROWS

    @pl.loop(0, 2 * PB_ROWS // L)
    def _(i):
        h = 2 * k0 + i * L + lax.iota(jnp.int32, L)
        k = h >> 1
        q = (k * 1311) >> 16
        r = k - q * NUM_BINS
        bidx_v[pl.ds(i * L, L)] = jnp.where(
            (h & 1) == 0,
            jnp.minimum(q, NUM_BINS - 1),
            NUM_BINS + r,
        )

    for o in range(0, 2 * PB_ROWS, 128):
        n = min(128, 2 * PB_ROWS - o)
        pltpu.async_copy(
            ct_hbm.at[bidx_v.at[pl.ds(o, n)]],
            hb.at[pl.ds(o, n)],
            gsems[0],
        ).wait()

    pltpu.sync_copy(hb, paired.at[pl.ds(k0, PB_ROWS)].reshape(2 * PB_ROWS, D_HALF))
    plsc.subcore_barrier()

    pltpu.sync_copy(xc_hbm.at[pl.ds(base, P_PER_W)], xc_v)
    pltpu.sync_copy(yc_hbm.at[pl.ds(base, P_PER_W)], yc_v)

    def bins(c):
        n = jnp.clip((c - R_MIN) / (R_MAX - R_MIN), 0.0, 1.0)
        return (n * float(NUM_BINS - 1)).astype(jnp.int32)

    @pl.loop(0, P_PER_W // L)
    def _(i):
        bx = bins(xc_v[pl.ds(i * L, L)])
        by = bins(yc_v[pl.ds(i * L, L)])
        kidx_v[pl.ds(i * L, L)] = bx * NUM_BINS + by

    def idx_slice(g):
        return kidx_v.at[pl.ds(g * CH, CH)]

    def out_slice(g):
        return out_hbm.at[pl.ds(base + g * CH, CH)]

    pltpu.async_copy(paired.at[idx_slice(0)], rbufs[0], gsems[0])
    pltpu.async_copy(paired.at[idx_slice(1)], rbufs[1], gsems[1])
    pltpu.async_copy(paired.at[idx_slice(2)], rbufs[2], gsems[2])

    @pl.loop(0, NCH, step=NBUF)
    def _(g0):
        for b in range(NBUF):
            g = g0 + b
            nb = (b + 3) % NBUF

            @pl.when(g + 3 < NCH)
            def _():
                @pl.when(g >= NBUF - 3)
                def _():
                    pltpu.make_async_copy(rbufs[nb], out_slice(0), wsems[nb]).wait()
                pltpu.async_copy(paired.at[idx_slice(g + 3)], rbufs[nb], gsems[nb])

            pltpu.make_async_copy(
                paired.at[idx_slice(g)], rbufs[b], gsems[b]).wait()
            pltpu.async_copy(rbufs[b], out_slice(g), wsems[b])

    for b in range(NBUF):
        pltpu.make_async_copy(rbufs[b], out_slice(0), wsems[b]).wait()


@jax.jit
def _pos_encode(xc, yc, comb):
    mesh = plsc.VectorSubcoreMesh(
        core_axis_name="c", subcore_axis_name="s", num_cores=NC, num_subcores=NS)
    f = pl.kernel(
        _sc_body,
        out_type=jax.ShapeDtypeStruct((N_POS, 2, D_HALF), jnp.float32),
        name="pos_encode_sc",
        mesh=mesh,
        scratch_types=[
            pltpu.VMEM((P_PER_W,), jnp.float32),
            pltpu.VMEM((P_PER_W,), jnp.float32),
            pltpu.VMEM((P_PER_W,), jnp.int32),
            pltpu.VMEM((2 * PB_ROWS,), jnp.int32),
            pltpu.VMEM((2 * PB_ROWS, D_HALF), jnp.float32),
            pltpu.VMEM_SHARED((PAIR_PAD, 2, D_HALF), jnp.float32),
            pltpu.VMEM((CH, 2, D_HALF), jnp.float32),
            pltpu.VMEM((CH, 2, D_HALF), jnp.float32),
            pltpu.VMEM((CH, 2, D_HALF), jnp.float32),
            pltpu.VMEM((CH, 2, D_HALF), jnp.float32),
            pltpu.VMEM((CH, 2, D_HALF), jnp.float32),
            pltpu.VMEM((CH, 2, D_HALF), jnp.float32),
            pltpu.VMEM((CH, 2, D_HALF), jnp.float32),
            pltpu.VMEM((CH, 2, D_HALF), jnp.float32),
            pltpu.SemaphoreType.DMA,
            pltpu.SemaphoreType.DMA,
            pltpu.SemaphoreType.DMA,
            pltpu.SemaphoreType.DMA,
            pltpu.SemaphoreType.DMA,
            pltpu.SemaphoreType.DMA,
            pltpu.SemaphoreType.DMA,
            pltpu.SemaphoreType.DMA,
            pltpu.SemaphoreType.DMA,
            pltpu.SemaphoreType.DMA,
            pltpu.SemaphoreType.DMA,
            pltpu.SemaphoreType.DMA,
            pltpu.SemaphoreType.DMA,
            pltpu.SemaphoreType.DMA,
            pltpu.SemaphoreType.DMA,
            pltpu.SemaphoreType.DMA,
        ],
    )
    return f(xc, yc, comb)


def kernel(positions, x_embed, y_embed):
    xc = positions[..., 0].reshape(-1)
    yc = positions[..., 1].reshape(-1)
    comb = jnp.concatenate([x_embed, y_embed], axis=0)
    out = _pos_encode(xc, yc, comb)
    return out.reshape(positions.shape[0], positions.shape[1], D)


# speedup vs baseline: 1.1641x; 1.0032x over previous
import jax
import jax.numpy as jnp
from jax import lax
from jax.experimental import pallas as pl
from jax.experimental.pallas import tpu as pltpu
from jax.experimental.pallas import tpu_sc as plsc

D_HALF = 128
D = 2 * D_HALF
NUM_BINS = 50
R_MIN = -4000.0
R_MAX = 4000.0

NC, NS, L = 2, 16, 16
NW = NC * NS

N_POS = 16 * 8192
P_PER_W = N_POS // NW
CH = 32
NCH = P_PER_W // CH
NBUF = 4

PAIR_PAD = 2560
PB_ROWS = PAIR_PAD // NS
B_PAD = 512
Y_OFF = 256


def _sc_body(xc_hbm, yc_hbm, ct_hbm, out_hbm,
             xc_v, yc_v, kidx_v, bidx_v, hb, paired,
             rb0, rb1, rb2, rb3, gs0, gs1, gs2, gs3,
             ws0, ws1, ws2, ws3):
    rbufs = (rb0, rb1, rb2, rb3)
    gsems = (gs0, gs1, gs2, gs3)
    wsems = (ws0, ws1, ws2, ws3)

    cid = lax.axis_index("c")
    sid = lax.axis_index("s")
    wid = sid * NC + cid
    base = wid * P_PER_W

    k0 = sid * PB_ROWS

    @pl.loop(0, 2 * PB_ROWS // L)
    def _(i):
        h = 2 * k0 + i * L + lax.iota(jnp.int32, L)
        k = h >> 1
        q = (k * 1311) >> 16
        r = k - q * NUM_BINS
        bidx_v[pl.ds(i * L, L)] = jnp.where(
            (h & 1) == 0,
            jnp.minimum(q, NUM_BINS - 1),
            NUM_BINS + r,
        )

    for o in range(0, 2 * PB_ROWS, 128):
        n = min(128, 2 * PB_ROWS - o)
        pltpu.async_copy(
            ct_hbm.at[bidx_v.at[pl.ds(o, n)]],
            hb.at[pl.ds(o, n)],
            gsems[0],
        ).wait()

    pltpu.sync_copy(hb, paired.at[pl.ds(k0, PB_ROWS)].reshape(2 * PB_ROWS, D_HALF))
    plsc.subcore_barrier()

    pltpu.sync_copy(xc_hbm.at[pl.ds(base, P_PER_W)], xc_v)
    pltpu.sync_copy(yc_hbm.at[pl.ds(base, P_PER_W)], yc_v)

    def bins(c):
        n = jnp.clip((c - R_MIN) / (R_MAX - R_MIN), 0.0, 1.0)
        return (n * float(NUM_BINS - 1)).astype(jnp.int32)

    @pl.loop(0, P_PER_W // L)
    def _(i):
        bx = bins(xc_v[pl.ds(i * L, L)])
        by = bins(yc_v[pl.ds(i * L, L)])
        kidx_v[pl.ds(i * L, L)] = bx * NUM_BINS + by

    def idx_slice(g):
        return kidx_v.at[pl.ds(g * CH, CH)]

    def out_slice(g):
        return out_hbm.at[pl.ds(base + g * CH, CH)]

    pltpu.async_copy(paired.at[idx_slice(0)], rbufs[0], gsems[0])
    pltpu.async_copy(paired.at[idx_slice(1)], rbufs[1], gsems[1])

    @pl.loop(0, NCH, step=NBUF)
    def _(g0):
        for b in range(NBUF):
            g = g0 + b
            nb = (b + 2) % NBUF

            @pl.when(g + 2 < NCH)
            def _():
                @pl.when(g >= 2)
                def _():
                    pltpu.make_async_copy(rbufs[nb], out_slice(0), wsems[nb]).wait()
                pltpu.async_copy(paired.at[idx_slice(g + 2)], rbufs[nb], gsems[nb])

            pltpu.make_async_copy(
                paired.at[idx_slice(g)], rbufs[b], gsems[b]).wait()
            pltpu.async_copy(rbufs[b], out_slice(g), wsems[b])

    for b in range(NBUF):
        pltpu.make_async_copy(rbufs[b], out_slice(0), wsems[b]).wait()


@jax.jit
def _pos_encode(xc, yc, comb):
    mesh = plsc.VectorSubcoreMesh(
        core_axis_name="c", subcore_axis_name="s", num_cores=NC, num_subcores=NS)
    f = pl.kernel(
        _sc_body,
        out_type=jax.ShapeDtypeStruct((N_POS, 2, D_HALF), jnp.float32),
        name="pos_encode_sc",
        mesh=mesh,
        scratch_types=[
            pltpu.VMEM((P_PER_W,), jnp.float32),
            pltpu.VMEM((P_PER_W,), jnp.float32),
            pltpu.VMEM((P_PER_W,), jnp.int32),
            pltpu.VMEM((2 * PB_ROWS,), jnp.int32),
            pltpu.VMEM((2 * PB_ROWS, D_HALF), jnp.float32),
            pltpu.VMEM_SHARED((PAIR_PAD, 2, D_HALF), jnp.float32),
            pltpu.VMEM((CH, 2, D_HALF), jnp.float32),
            pltpu.VMEM((CH, 2, D_HALF), jnp.float32),
            pltpu.VMEM((CH, 2, D_HALF), jnp.float32),
            pltpu.VMEM((CH, 2, D_HALF), jnp.float32),
            pltpu.SemaphoreType.DMA,
            pltpu.SemaphoreType.DMA,
            pltpu.SemaphoreType.DMA,
            pltpu.SemaphoreType.DMA,
            pltpu.SemaphoreType.DMA,
            pltpu.SemaphoreType.DMA,
            pltpu.SemaphoreType.DMA,
            pltpu.SemaphoreType.DMA,
        ],
    )
    return f(xc, yc, comb)


def kernel(positions, x_embed, y_embed):
    xc = positions[..., 0].reshape(-1)
    yc = positions[..., 1].reshape(-1)
    comb = jnp.concatenate([x_embed, y_embed], axis=0)
    out = _pos_encode(xc, yc, comb)
    return out.reshape(positions.shape[0], positions.shape[1], D)
